# baseline (speedup 1.0000x reference)
import functools

import jax
import jax.numpy as jnp
from jax import lax
from jax.experimental import pallas as pl
from jax.experimental.pallas import tpu as pltpu
from jax.experimental.pallas import tpu_sc as plsc

N = 10000
E = 160000
TW = 25
H = 128
NL = 6
L_PDE = 16.0
TMAX = 4.0
DT = 0.16

F32 = jnp.float32

NC = 2
NS = 16
NW = NC * NS
CH = 128
CHN = E // CH
CHP = 1280
EP = CHP * CH
GB = CHN // NW
GX = CHN % NW
GBU = CHP // NW
SBU = CHP // NS
DTN = 10
DRW = N // DTN
ZRW = 40

def _swish(x):
    return x * (1.0 / (1.0 + jnp.exp(-x)))


def _sigmoid(x):
    return 1.0 / (1.0 + jnp.exp(-x))


@functools.cache
def _sc_kernels():
    mesh = plsc.VectorSubcoreMesh(core_axis_name="c", subcore_axis_name="s",
                                  num_cores=NC, num_subcores=NS)
    gather = functools.partial(
        pl.kernel,
        out_type=jax.ShapeDtypeStruct((EP, 2 * H), F32),
        mesh=mesh,
        scratch_types=[
            pltpu.VMEM((GBU, 1, CH), jnp.int32),
            pltpu.VMEM((GBU, 1, CH), jnp.int32),
            pltpu.VMEM((CH, 2 * H), F32),
            pltpu.VMEM((CH, 2 * H), F32),
            pltpu.SemaphoreType.DMA,
            pltpu.SemaphoreType.DMA,
        ],
    )(_gather_body)
    scatter = functools.partial(
        pl.kernel,
        out_type=jax.ShapeDtypeStruct((NC, N, H), F32),
        mesh=mesh,
        scratch_types=[
            pltpu.VMEM((SBU, 1, CH), jnp.int32),
            pltpu.VMEM((CH, H), F32),
            pltpu.VMEM((ZRW, H), F32),
            pltpu.VMEM_SHARED((N, H), F32),
        ],
    )(_scatter_body)
    degree = functools.partial(
        pl.kernel,
        out_type=jax.ShapeDtypeStruct((NC, N, H), F32),
        mesh=mesh,
        scratch_types=[
            pltpu.VMEM((GB + 1, 1, CH), jnp.int32),
            pltpu.VMEM((CH, H), F32),
            pltpu.VMEM((ZRW, H), F32),
            pltpu.VMEM_SHARED((N, H), F32),
        ],
    )(_degree_body)
    return gather, scatter, degree


def _gather_pairs(a_tab, b_tab, dst2, src2):
    return _sc_kernels()[0](a_tab, b_tab, dst2, src2)


def _scatter_sum(m_edge, dst2):
    return _sc_kernels()[1](m_edge, dst2)


def _degree(dst2):
    return _sc_kernels()[2](dst2)


def _gather_body(a_hbm, b_hbm, dst_hbm, src_hbm, out_hbm,
                 dst_v, src_v, ra, rb, sem_a, sem_b):
    wid = lax.axis_index("s") * NC + lax.axis_index("c")
    start = GBU * wid
    pltpu.sync_copy(dst_hbm.at[pl.ds(start, GBU)], dst_v)
    pltpu.sync_copy(src_hbm.at[pl.ds(start, GBU)], src_v)

    def body(j, carry):
        ca = pltpu.async_copy(a_hbm.at[dst_v.at[j, 0]], ra, sem_a)
        cb = pltpu.async_copy(b_hbm.at[src_v.at[j, 0]], rb, sem_b)
        ca.wait()
        cb.wait()

        def add_row(r, c):
            for blk in range(2 * H // 16):
                sl = pl.ds(blk * 16, 16)
                ra[r, sl] = ra[r, sl] + rb[r, sl]
            return c

        lax.fori_loop(0, CH, add_row, 0)
        pltpu.sync_copy(ra, out_hbm.at[pl.ds((start + j) * CH, CH), :])
        return carry

    lax.fori_loop(0, GBU, body, 0)


def _scatter_body(m_hbm, dst_hbm, out_hbm, dst_v, m0, zbuf, acc):
    cid = lax.axis_index("c")
    sid = lax.axis_index("s")
    start = SBU * sid
    pltpu.sync_copy(dst_hbm.at[pl.ds(start, SBU)], dst_v)

    def zrow(r, c):
        for blk in range(H // 16):
            zbuf[r, pl.ds(blk * 16, 16)] = jnp.zeros((16,), F32)
        return c

    lax.fori_loop(0, ZRW, zrow, 0)

    @pl.when(sid < DTN)
    def _():
        for k in range(DRW // ZRW):
            pltpu.sync_copy(zbuf, acc.at[pl.ds(sid * DRW + k * ZRW, ZRW), :])

    plsc.subcore_barrier()

    def body(j, carry):
        pltpu.sync_copy(
            m_hbm.at[pl.ds((start + j) * CH, CH), pl.ds(cid * H, H)], m0)
        pltpu.sync_copy(m0, acc.at[dst_v.at[j, 0]], add=True)
        return carry

    lax.fori_loop(0, SBU, body, 0)
    plsc.subcore_barrier()

    @pl.when(sid < DTN)
    def _():
        sl = pl.ds(sid * DRW, DRW)
        pltpu.sync_copy(acc.at[sl, :], out_hbm.at[cid, sl, :])


def _degree_body(dst_hbm, out_hbm, dst_v, ones_v, zbuf, acc):
    cid = lax.axis_index("c")
    sid = lax.axis_index("s")
    wid = sid * NC + cid
    start = GB * wid + jnp.minimum(wid, GX)
    nch = GB + jnp.where(wid < GX, 1, 0)
    pltpu.sync_copy(dst_hbm.at[pl.ds(start, GB + 1)], dst_v)

    def fill(r, c):
        for blk in range(H // 16):
            ones_v[r, pl.ds(blk * 16, 16)] = jnp.ones((16,), F32)
        return c

    lax.fori_loop(0, CH, fill, 0)

    def zfill(r, c):
        for blk in range(H // 16):
            zbuf[r, pl.ds(blk * 16, 16)] = jnp.zeros((16,), F32)
        return c

    lax.fori_loop(0, ZRW, zfill, 0)

    @pl.when(sid < DTN)
    def _():
        for k in range(DRW // ZRW):
            pltpu.sync_copy(zbuf, acc.at[pl.ds(sid * DRW + k * ZRW, ZRW), :])

    plsc.subcore_barrier()

    def body(j, carry):
        pltpu.sync_copy(ones_v, acc.at[dst_v.at[j, 0]], add=True)
        return carry

    lax.fori_loop(0, nch, body, 0)
    plsc.subcore_barrier()

    @pl.when(sid < DTN)
    def _():
        sl = pl.ds(sid * DRW, DRW)
        pltpu.sync_copy(acc.at[sl, :], out_hbm.at[cid, sl, :])


def _embed_body(z_ref, cntp_ref, w1_ref, b1_ref, w2_ref, b2_ref,
                h_ref, cinv_ref):
    z = z_ref[...]
    h0 = _swish(jnp.dot(z, w1_ref[...], preferred_element_type=F32)
                + b1_ref[...])
    h_ref[...] = _swish(jnp.dot(h0, w2_ref[...], preferred_element_type=F32)
                        + b2_ref[...])
    cnt = cntp_ref[0, :, 0:1] + cntp_ref[1, :, 0:1]
    cinv_ref[...] = 1.0 / jnp.maximum(cnt, 1.0)


def _embed(z, cntp, w1, b1, w2, b2):
    return pl.pallas_call(
        _embed_body,
        out_shape=[jax.ShapeDtypeStruct((N, H), F32),
                   jax.ShapeDtypeStruct((N, 1), F32)],
    )(z, cntp, w1, b1, w2, b2)


NBK = 1000


def _prep_body(h_ref, z_ref, whA_ref, whB_ref, wzA_ref, wzB_ref, bA_ref,
               a_ref, b_ref):
    h = h_ref[...]
    z = z_ref[...]
    a_ref[...] = (jnp.dot(h, whA_ref[...], preferred_element_type=F32)
                  + jnp.dot(z, wzA_ref[...], preferred_element_type=F32)
                  + bA_ref[...])
    b_ref[...] = (jnp.dot(h, whB_ref[...], preferred_element_type=F32)
                  + jnp.dot(z, wzB_ref[...], preferred_element_type=F32))


def _prep(h, z, whA, whB, wzA, wzB, bA):
    nb = N // NBK
    blk = lambda i: (i, 0)
    zero = lambda i: (0, 0)
    return pl.pallas_call(
        _prep_body,
        grid=(nb,),
        in_specs=[
            pl.BlockSpec((NBK, H), blk),
            pl.BlockSpec((NBK, 52), blk),
            pl.BlockSpec((H, 2 * H), zero),
            pl.BlockSpec((H, 2 * H), zero),
            pl.BlockSpec((52, 2 * H), zero),
            pl.BlockSpec((52, 2 * H), zero),
            pl.BlockSpec((1, 2 * H), zero),
        ],
        out_specs=[pl.BlockSpec((NBK, 2 * H), blk),
                   pl.BlockSpec((NBK, 2 * H), blk)],
        out_shape=[jax.ShapeDtypeStruct((N, 2 * H), F32),
                   jax.ShapeDtypeStruct((N, 2 * H), F32)],
    )(h, z, whA, whB, wzA, wzB, bA)


EBK = 1024


def _edge_body(p_ref, wg_ref, bg_ref, wn_ref, bn_ref, m_ref):
    rows = (pl.program_id(0) * EBK
            + lax.broadcasted_iota(jnp.int32, (EBK, 1), 0))
    mask = rows < E
    s = _swish(p_ref[...]).astype(jnp.bfloat16)
    mg = _swish(jnp.dot(s[:, :H], wg_ref[...], preferred_element_type=F32)
                + bg_ref[...])
    mn = _swish(jnp.dot(s[:, H:], wn_ref[...], preferred_element_type=F32)
                + bn_ref[...])
    m_ref[:, :H] = jnp.where(mask, mg, 0.0)
    m_ref[:, H:] = jnp.where(mask, mn, 0.0)


def _edge_mlp(p, wg, bg, wn, bn):
    nb = EP // EBK
    blk = lambda i: (i, 0)
    zero = lambda i: (0, 0)
    return pl.pallas_call(
        _edge_body,
        grid=(nb,),
        in_specs=[
            pl.BlockSpec((EBK, 2 * H), blk),
            pl.BlockSpec((H, H), zero),
            pl.BlockSpec((1, H), zero),
            pl.BlockSpec((H, H), zero),
            pl.BlockSpec((1, H), zero),
        ],
        out_specs=pl.BlockSpec((EBK, 2 * H), blk),
        out_shape=jax.ShapeDtypeStruct((EP, 2 * H), F32),
    )(p, wg, bg, wn, bn)


def _update_body(h_ref, ag_ref, an_ref, cinv_ref, var_ref,
                 whg_ref, wag_ref, wvg_ref, b1g_ref, w2g_ref, b2g_ref,
                 whn_ref, wan_ref, wvn_ref, b1n_ref, w2n_ref, b2n_ref,
                 hn_ref):
    h = h_ref[...]
    cinv = cinv_ref[...]
    var = var_ref[...]

    def half(ag, wh, wa, wv, b1, w2, b2):
        mean = ag * cinv
        t = _swish(jnp.dot(h, wh, preferred_element_type=F32)
                   + jnp.dot(mean, wa, preferred_element_type=F32)
                   + var * wv + b1)
        upd = jnp.dot(t, w2, preferred_element_type=F32) + b2
        out = h + upd
        mu = jnp.mean(out, axis=0, keepdims=True)
        d = out - mu
        v = jnp.mean(d * d, axis=0, keepdims=True)
        return d * lax.rsqrt(v + 1e-5)

    ngate = half(ag_ref[...], whg_ref[...], wag_ref[...], wvg_ref[...],
                 b1g_ref[...], w2g_ref[...], b2g_ref[...])
    ngnn = half(an_ref[...], whn_ref[...], wan_ref[...], wvn_ref[...],
                b1n_ref[...], w2n_ref[...], b2n_ref[...])
    tau = _sigmoid(ngate)
    g = _swish(ngnn)
    hn_ref[...] = (1.0 - tau) * h + tau * g


def _update(h, ag, an, cinv, var, wts):
    return pl.pallas_call(
        _update_body,
        out_shape=jax.ShapeDtypeStruct((N, H), F32),
    )(h, ag, an, cinv, var, *wts)


NC1 = 38
K1 = 16
S1 = 3
K2 = 14


def _dec_body(h_ref, u_ref, wd_ref, bd_ref, w1_ref, b1_ref, w2_ref, b2_ref,
              o_ref, d1_ref):
    hd = _swish(jnp.dot(h_ref[...], wd_ref[...], preferred_element_type=F32)
                + bd_ref[...])
    for t in range(NC1):
        x1 = jnp.concatenate(
            [hd[:, S1 * t:S1 * t + K1], hd[:, H + S1 * t:H + S1 * t + K1]],
            axis=1)
        d1_ref[:, t * 8:(t + 1) * 8] = _swish(
            jnp.dot(x1, w1_ref[...], preferred_element_type=F32) + b1_ref[...])
    for t in range(TW):
        x2 = d1_ref[:, t * 8:t * 8 + 8 * K2]
        d2 = jnp.dot(x2, w2_ref[...], preferred_element_type=F32) + b2_ref[...]
        dt = DT * (t + 1)
        o_ref[:, pl.ds(t, 1)] = u_ref[:, pl.ds(t, 1)] + dt * d2[:, 0:1]
        o_ref[:, pl.ds(TW + t, 1)] = u_ref[:, pl.ds(TW + t, 1)] + dt * d2[:, 1:2]


def _decode(h, u, wd, bd, w1, b1, w2, b2):
    nb = N // NBK
    blk = lambda i: (i, 0)
    zero = lambda i: (0, 0)
    return pl.pallas_call(
        _dec_body,
        grid=(nb,),
        in_specs=[
            pl.BlockSpec((NBK, H), blk),
            pl.BlockSpec((NBK, 2 * TW), blk),
            pl.BlockSpec((H, 2 * H), zero),
            pl.BlockSpec((1, 2 * H), zero),
            pl.BlockSpec((2 * K1, 8), zero),
            pl.BlockSpec((1, 8), zero),
            pl.BlockSpec((8 * K2, 2), zero),
            pl.BlockSpec((1, 2), zero),
        ],
        out_specs=pl.BlockSpec((NBK, 2 * TW), blk),
        out_shape=jax.ShapeDtypeStruct((N, 2 * TW), F32),
        scratch_shapes=[pltpu.VMEM((NBK, 8 * NC1), F32)],
    )(h, u, wd, bd, w1, b1, w2, b2)


def kernel(x, pos, edge_index, batch, params):
    del batch
    pad = jnp.zeros((CHP - CHN, CH), jnp.int32)
    src2 = jnp.concatenate([edge_index[0].reshape(CHN, CH), pad]
                           ).reshape(CHP, 1, CH)
    dst2 = jnp.concatenate([edge_index[1].reshape(CHN, CH), pad]
                           ).reshape(CHP, 1, CH)
    pos_x = pos[:, 1:2] / L_PDE
    variables = pos[:, 0:1] / TMAX
    z = jnp.concatenate([x, pos_x, variables], axis=1)

    w1e = params["emb1"]["W"].T
    b1e = params["emb1"]["b"].reshape(1, H)
    w2e = params["emb2"]["W"].T
    b2e = params["emb2"]["b"].reshape(1, H)

    layer_w = []
    for i in range(NL):
        pg = params["gate"][i]
        pn = params["gnn"][i]
        Wg = pg["msg1"]["W"]
        Wn = pn["msg1"]["W"]
        whA = jnp.concatenate([Wg[:, :H].T, Wn[:, :H].T], axis=1)
        whB = jnp.concatenate([Wg[:, H:2 * H].T, Wn[:, H:2 * H].T], axis=1)
        wzA = jnp.concatenate([Wg[:, 2 * H:].T, Wn[:, 2 * H:].T], axis=1)
        mask = jnp.ones((52, 1), F32).at[51, 0].set(0.0)
        wzB = jnp.concatenate([-Wg[:, 2 * H:].T * mask,
                               -Wn[:, 2 * H:].T * mask], axis=1)
        bA = jnp.concatenate([pg["msg1"]["b"], pn["msg1"]["b"]]).reshape(1, 2 * H)
        w2g = pg["msg2"]["W"].T.astype(jnp.bfloat16)
        b2g = pg["msg2"]["b"].reshape(1, H)
        w2n = pn["msg2"]["W"].T.astype(jnp.bfloat16)
        b2n = pn["msg2"]["b"].reshape(1, H)

        def upd_w(p):
            U1 = p["upd1"]["W"]
            return (U1[:, :H].T, U1[:, H:2 * H].T, U1[:, 2 * H:].T,
                    p["upd1"]["b"].reshape(1, H), p["upd2"]["W"].T,
                    p["upd2"]["b"].reshape(1, H))

        layer_w.append((whA, whB, wzA, wzB, bA, w2g, b2g, w2n, b2n,
                        upd_w(pg) + upd_w(pn)))

    wd = params["double"]["W"].T
    bd = params["double"]["b"].reshape(1, 2 * H)
    w1c = jnp.transpose(params["conv1"]["W"], (1, 2, 0)).reshape(2 * K1, 8)
    b1c = params["conv1"]["b"].reshape(1, 8)
    w2c = jnp.transpose(params["conv2"]["W"], (2, 1, 0)).reshape(8 * K2, 2)
    b2c = params["conv2"]["b"].reshape(1, 2)

    cntp = _degree(dst2)
    h, cinv = _embed(z, cntp, w1e, b1e, w2e, b2e)
    for i in range(NL):
        whA, whB, wzA, wzB, bA, w2g, b2g, w2n, b2n, uw = layer_w[i]
        a_tab, b_tab = _prep(h, z, whA, whB, wzA, wzB, bA)
        p_edge = _gather_pairs(a_tab, b_tab, dst2, src2)
        m_edge = _edge_mlp(p_edge, w2g, b2g, w2n, b2n)
        agg = _scatter_sum(m_edge, dst2)
        h = _update(h, agg[0], agg[1], cinv, variables, uw)
    return _decode(h, x, wd, bd, w1c, b1c, w2c, b2c)


# speedup vs baseline: 1.0739x; 1.0739x over previous
import functools

import jax
import jax.numpy as jnp
from jax import lax
from jax.experimental import pallas as pl
from jax.experimental.pallas import tpu as pltpu
from jax.experimental.pallas import tpu_sc as plsc

N = 10000
E = 160000
TW = 25
H = 128
NL = 6
L_PDE = 16.0
TMAX = 4.0
DT = 0.16

F32 = jnp.float32

NC = 2
NS = 16
NW = NC * NS
CH = 128
CHN = E // CH
CHP = 1280
EP = CHP * CH
GB = CHN // NW
GX = CHN % NW
GBU = CHP // NW
SBU = CHP // NS
DTN = 10
DRW = N // DTN
ZRW = 40

def _swish(x):
    return x * (1.0 / (1.0 + jnp.exp(-x)))


def _sigmoid(x):
    return 1.0 / (1.0 + jnp.exp(-x))


@functools.cache
def _sc_kernels():
    mesh = plsc.VectorSubcoreMesh(core_axis_name="c", subcore_axis_name="s",
                                  num_cores=NC, num_subcores=NS)
    gather = functools.partial(
        pl.kernel,
        out_type=jax.ShapeDtypeStruct((EP, 2 * H), F32),
        mesh=mesh,
        scratch_types=[
            pltpu.VMEM((GBU, 1, CH), jnp.int32),
            pltpu.VMEM((GBU, 1, CH), jnp.int32),
            pltpu.VMEM((CH, 2 * H), F32),
            pltpu.VMEM((CH, 2 * H), F32),
            pltpu.SemaphoreType.DMA,
            pltpu.SemaphoreType.DMA,
        ],
    )(_gather_body)
    scatter = functools.partial(
        pl.kernel,
        out_type=jax.ShapeDtypeStruct((NC, N, H), F32),
        mesh=mesh,
        scratch_types=[
            pltpu.VMEM((SBU, 1, CH), jnp.int32),
            pltpu.VMEM((CH, H), F32),
            pltpu.VMEM((CH, H), F32),
            pltpu.VMEM((ZRW, H), F32),
            pltpu.VMEM_SHARED((N, H), F32),
            pltpu.SemaphoreType.DMA,
            pltpu.SemaphoreType.DMA,
        ],
    )(_scatter_body)
    degree = functools.partial(
        pl.kernel,
        out_type=jax.ShapeDtypeStruct((NC, N, H), F32),
        mesh=mesh,
        scratch_types=[
            pltpu.VMEM((GB + 1, 1, CH), jnp.int32),
            pltpu.VMEM((CH, H), F32),
            pltpu.VMEM((ZRW, H), F32),
            pltpu.VMEM_SHARED((N, H), F32),
        ],
    )(_degree_body)
    return gather, scatter, degree


def _gather_pairs(a_tab, b_tab, dst2, src2):
    return _sc_kernels()[0](a_tab, b_tab, dst2, src2)


def _scatter_sum(m_edge, dst2):
    return _sc_kernels()[1](m_edge, dst2)


def _degree(dst2):
    return _sc_kernels()[2](dst2)


def _gather_body(a_hbm, b_hbm, dst_hbm, src_hbm, out_hbm,
                 dst_v, src_v, ra, rb, sem_a, sem_b):
    wid = lax.axis_index("s") * NC + lax.axis_index("c")
    start = GBU * wid
    pltpu.sync_copy(dst_hbm.at[pl.ds(start, GBU)], dst_v)
    pltpu.sync_copy(src_hbm.at[pl.ds(start, GBU)], src_v)

    def body(j, carry):
        ca = pltpu.async_copy(a_hbm.at[dst_v.at[j, 0]], ra, sem_a)
        cb = pltpu.async_copy(b_hbm.at[src_v.at[j, 0]], rb, sem_b)
        ca.wait()
        cb.wait()

        def add_row(r, c):
            for blk in range(2 * H // 16):
                sl = pl.ds(blk * 16, 16)
                ra[r, sl] = ra[r, sl] + rb[r, sl]
            return c

        lax.fori_loop(0, CH, add_row, 0)
        pltpu.sync_copy(ra, out_hbm.at[pl.ds((start + j) * CH, CH), :])
        return carry

    lax.fori_loop(0, GBU, body, 0)


def _scatter_body(m_hbm, dst_hbm, out_hbm, dst_v, m0, m1, zbuf, acc,
                  sem0, sem1):
    cid = lax.axis_index("c")
    sid = lax.axis_index("s")
    start = SBU * sid
    pltpu.sync_copy(dst_hbm.at[pl.ds(start, SBU)], dst_v)

    def zrow(r, c):
        for blk in range(H // 16):
            zbuf[r, pl.ds(blk * 16, 16)] = jnp.zeros((16,), F32)
        return c

    lax.fori_loop(0, ZRW, zrow, 0)

    @pl.when(sid < DTN)
    def _():
        for k in range(DRW // ZRW):
            pltpu.sync_copy(zbuf, acc.at[pl.ds(sid * DRW + k * ZRW, ZRW), :])

    plsc.subcore_barrier()

    def load(j, buf, sem):
        return pltpu.async_copy(
            m_hbm.at[pl.ds((start + j) * CH, CH), pl.ds(cid * H, H)], buf, sem)

    def drain(buf, sem):
        pltpu.make_async_copy(
            m_hbm.at[pl.ds(0, CH), pl.ds(0, H)], buf, sem).wait()

    load(0, m0, sem0)
    npairs = SBU // 2

    def body(jp, carry):
        j0 = 2 * jp
        load(j0 + 1, m1, sem1)
        drain(m0, sem0)
        pltpu.sync_copy(m0, acc.at[dst_v.at[j0, 0]], add=True)

        @pl.when(jp + 1 < npairs)
        def _():
            load(j0 + 2, m0, sem0)

        drain(m1, sem1)
        pltpu.sync_copy(m1, acc.at[dst_v.at[j0 + 1, 0]], add=True)
        return carry

    lax.fori_loop(0, npairs, body, 0)
    plsc.subcore_barrier()

    @pl.when(sid < DTN)
    def _():
        sl = pl.ds(sid * DRW, DRW)
        pltpu.sync_copy(acc.at[sl, :], out_hbm.at[cid, sl, :])


def _degree_body(dst_hbm, out_hbm, dst_v, ones_v, zbuf, acc):
    cid = lax.axis_index("c")
    sid = lax.axis_index("s")
    wid = sid * NC + cid
    start = GB * wid + jnp.minimum(wid, GX)
    nch = GB + jnp.where(wid < GX, 1, 0)
    pltpu.sync_copy(dst_hbm.at[pl.ds(start, GB + 1)], dst_v)

    def fill(r, c):
        for blk in range(H // 16):
            ones_v[r, pl.ds(blk * 16, 16)] = jnp.ones((16,), F32)
        return c

    lax.fori_loop(0, CH, fill, 0)

    def zfill(r, c):
        for blk in range(H // 16):
            zbuf[r, pl.ds(blk * 16, 16)] = jnp.zeros((16,), F32)
        return c

    lax.fori_loop(0, ZRW, zfill, 0)

    @pl.when(sid < DTN)
    def _():
        for k in range(DRW // ZRW):
            pltpu.sync_copy(zbuf, acc.at[pl.ds(sid * DRW + k * ZRW, ZRW), :])

    plsc.subcore_barrier()

    def body(j, carry):
        pltpu.sync_copy(ones_v, acc.at[dst_v.at[j, 0]], add=True)
        return carry

    lax.fori_loop(0, nch, body, 0)
    plsc.subcore_barrier()

    @pl.when(sid < DTN)
    def _():
        sl = pl.ds(sid * DRW, DRW)
        pltpu.sync_copy(acc.at[sl, :], out_hbm.at[cid, sl, :])


def _embed_body(z_ref, cntp_ref, w1_ref, b1_ref, w2_ref, b2_ref,
                h_ref, cinv_ref):
    z = z_ref[...]
    h0 = _swish(jnp.dot(z, w1_ref[...], preferred_element_type=F32)
                + b1_ref[...])
    h_ref[...] = _swish(jnp.dot(h0, w2_ref[...], preferred_element_type=F32)
                        + b2_ref[...])
    cnt = cntp_ref[0, :, 0:1] + cntp_ref[1, :, 0:1]
    cinv_ref[...] = 1.0 / jnp.maximum(cnt, 1.0)


def _embed(z, cntp, w1, b1, w2, b2):
    return pl.pallas_call(
        _embed_body,
        out_shape=[jax.ShapeDtypeStruct((N, H), F32),
                   jax.ShapeDtypeStruct((N, 1), F32)],
    )(z, cntp, w1, b1, w2, b2)


NBK = 1000


def _prep_body(h_ref, z_ref, whA_ref, whB_ref, wzA_ref, wzB_ref, bA_ref,
               a_ref, b_ref):
    h = h_ref[...]
    z = z_ref[...]
    a_ref[...] = (jnp.dot(h, whA_ref[...], preferred_element_type=F32)
                  + jnp.dot(z, wzA_ref[...], preferred_element_type=F32)
                  + bA_ref[...])
    b_ref[...] = (jnp.dot(h, whB_ref[...], preferred_element_type=F32)
                  + jnp.dot(z, wzB_ref[...], preferred_element_type=F32))


def _prep(h, z, whA, whB, wzA, wzB, bA):
    nb = N // NBK
    blk = lambda i: (i, 0)
    zero = lambda i: (0, 0)
    return pl.pallas_call(
        _prep_body,
        grid=(nb,),
        in_specs=[
            pl.BlockSpec((NBK, H), blk),
            pl.BlockSpec((NBK, 52), blk),
            pl.BlockSpec((H, 2 * H), zero),
            pl.BlockSpec((H, 2 * H), zero),
            pl.BlockSpec((52, 2 * H), zero),
            pl.BlockSpec((52, 2 * H), zero),
            pl.BlockSpec((1, 2 * H), zero),
        ],
        out_specs=[pl.BlockSpec((NBK, 2 * H), blk),
                   pl.BlockSpec((NBK, 2 * H), blk)],
        out_shape=[jax.ShapeDtypeStruct((N, 2 * H), F32),
                   jax.ShapeDtypeStruct((N, 2 * H), F32)],
    )(h, z, whA, whB, wzA, wzB, bA)


EBK = 1024


def _edge_body(p_ref, wg_ref, bg_ref, wn_ref, bn_ref, m_ref):
    rows = (pl.program_id(0) * EBK
            + lax.broadcasted_iota(jnp.int32, (EBK, 1), 0))
    mask = rows < E
    s = _swish(p_ref[...])
    mg = _swish(jnp.dot(s[:, :H], wg_ref[...], preferred_element_type=F32)
                + bg_ref[...])
    mn = _swish(jnp.dot(s[:, H:], wn_ref[...], preferred_element_type=F32)
                + bn_ref[...])
    m_ref[:, :H] = jnp.where(mask, mg, 0.0)
    m_ref[:, H:] = jnp.where(mask, mn, 0.0)


def _edge_mlp(p, wg, bg, wn, bn):
    nb = EP // EBK
    blk = lambda i: (i, 0)
    zero = lambda i: (0, 0)
    return pl.pallas_call(
        _edge_body,
        grid=(nb,),
        in_specs=[
            pl.BlockSpec((EBK, 2 * H), blk),
            pl.BlockSpec((H, H), zero),
            pl.BlockSpec((1, H), zero),
            pl.BlockSpec((H, H), zero),
            pl.BlockSpec((1, H), zero),
        ],
        out_specs=pl.BlockSpec((EBK, 2 * H), blk),
        out_shape=jax.ShapeDtypeStruct((EP, 2 * H), F32),
    )(p, wg, bg, wn, bn)


def _update_body(h_ref, ag_ref, an_ref, cinv_ref, var_ref,
                 whg_ref, wag_ref, wvg_ref, b1g_ref, w2g_ref, b2g_ref,
                 whn_ref, wan_ref, wvn_ref, b1n_ref, w2n_ref, b2n_ref,
                 hn_ref):
    h = h_ref[...]
    cinv = cinv_ref[...]
    var = var_ref[...]

    def half(ag, wh, wa, wv, b1, w2, b2):
        mean = ag * cinv
        t = _swish(jnp.dot(h, wh, preferred_element_type=F32)
                   + jnp.dot(mean, wa, preferred_element_type=F32)
                   + var * wv + b1)
        upd = jnp.dot(t, w2, preferred_element_type=F32) + b2
        out = h + upd
        mu = jnp.mean(out, axis=0, keepdims=True)
        d = out - mu
        v = jnp.mean(d * d, axis=0, keepdims=True)
        return d * lax.rsqrt(v + 1e-5)

    ngate = half(ag_ref[...], whg_ref[...], wag_ref[...], wvg_ref[...],
                 b1g_ref[...], w2g_ref[...], b2g_ref[...])
    ngnn = half(an_ref[...], whn_ref[...], wan_ref[...], wvn_ref[...],
                b1n_ref[...], w2n_ref[...], b2n_ref[...])
    tau = _sigmoid(ngate)
    g = _swish(ngnn)
    hn_ref[...] = (1.0 - tau) * h + tau * g


def _update(h, ag, an, cinv, var, wts):
    return pl.pallas_call(
        _update_body,
        out_shape=jax.ShapeDtypeStruct((N, H), F32),
    )(h, ag, an, cinv, var, *wts)


NC1 = 38
K1 = 16
S1 = 3
K2 = 14


def _dec_body(h_ref, u_ref, wd_ref, bd_ref, w1_ref, b1_ref, w2_ref, b2_ref,
              o_ref, d1_ref):
    hd = _swish(jnp.dot(h_ref[...], wd_ref[...], preferred_element_type=F32)
                + bd_ref[...])
    for t in range(NC1):
        x1 = jnp.concatenate(
            [hd[:, S1 * t:S1 * t + K1], hd[:, H + S1 * t:H + S1 * t + K1]],
            axis=1)
        d1_ref[:, t * 8:(t + 1) * 8] = _swish(
            jnp.dot(x1, w1_ref[...], preferred_element_type=F32) + b1_ref[...])
    for t in range(TW):
        x2 = d1_ref[:, t * 8:t * 8 + 8 * K2]
        d2 = jnp.dot(x2, w2_ref[...], preferred_element_type=F32) + b2_ref[...]
        dt = DT * (t + 1)
        o_ref[:, pl.ds(t, 1)] = u_ref[:, pl.ds(t, 1)] + dt * d2[:, 0:1]
        o_ref[:, pl.ds(TW + t, 1)] = u_ref[:, pl.ds(TW + t, 1)] + dt * d2[:, 1:2]


def _decode(h, u, wd, bd, w1, b1, w2, b2):
    nb = N // NBK
    blk = lambda i: (i, 0)
    zero = lambda i: (0, 0)
    return pl.pallas_call(
        _dec_body,
        grid=(nb,),
        in_specs=[
            pl.BlockSpec((NBK, H), blk),
            pl.BlockSpec((NBK, 2 * TW), blk),
            pl.BlockSpec((H, 2 * H), zero),
            pl.BlockSpec((1, 2 * H), zero),
            pl.BlockSpec((2 * K1, 8), zero),
            pl.BlockSpec((1, 8), zero),
            pl.BlockSpec((8 * K2, 2), zero),
            pl.BlockSpec((1, 2), zero),
        ],
        out_specs=pl.BlockSpec((NBK, 2 * TW), blk),
        out_shape=jax.ShapeDtypeStruct((N, 2 * TW), F32),
        scratch_shapes=[pltpu.VMEM((NBK, 8 * NC1), F32)],
    )(h, u, wd, bd, w1, b1, w2, b2)


def kernel(x, pos, edge_index, batch, params):
    del batch
    pad = jnp.zeros((CHP - CHN, CH), jnp.int32)
    src2 = jnp.concatenate([edge_index[0].reshape(CHN, CH), pad]
                           ).reshape(CHP, 1, CH)
    dst2 = jnp.concatenate([edge_index[1].reshape(CHN, CH), pad]
                           ).reshape(CHP, 1, CH)
    pos_x = pos[:, 1:2] / L_PDE
    variables = pos[:, 0:1] / TMAX
    z = jnp.concatenate([x, pos_x, variables], axis=1)

    w1e = params["emb1"]["W"].T
    b1e = params["emb1"]["b"].reshape(1, H)
    w2e = params["emb2"]["W"].T
    b2e = params["emb2"]["b"].reshape(1, H)

    layer_w = []
    for i in range(NL):
        pg = params["gate"][i]
        pn = params["gnn"][i]
        Wg = pg["msg1"]["W"]
        Wn = pn["msg1"]["W"]
        whA = jnp.concatenate([Wg[:, :H].T, Wn[:, :H].T], axis=1)
        whB = jnp.concatenate([Wg[:, H:2 * H].T, Wn[:, H:2 * H].T], axis=1)
        wzA = jnp.concatenate([Wg[:, 2 * H:].T, Wn[:, 2 * H:].T], axis=1)
        mask = jnp.ones((52, 1), F32).at[51, 0].set(0.0)
        wzB = jnp.concatenate([-Wg[:, 2 * H:].T * mask,
                               -Wn[:, 2 * H:].T * mask], axis=1)
        bA = jnp.concatenate([pg["msg1"]["b"], pn["msg1"]["b"]]).reshape(1, 2 * H)
        w2g = pg["msg2"]["W"].T
        b2g = pg["msg2"]["b"].reshape(1, H)
        w2n = pn["msg2"]["W"].T
        b2n = pn["msg2"]["b"].reshape(1, H)

        def upd_w(p):
            U1 = p["upd1"]["W"]
            return (U1[:, :H].T, U1[:, H:2 * H].T, U1[:, 2 * H:].T,
                    p["upd1"]["b"].reshape(1, H), p["upd2"]["W"].T,
                    p["upd2"]["b"].reshape(1, H))

        layer_w.append((whA, whB, wzA, wzB, bA, w2g, b2g, w2n, b2n,
                        upd_w(pg) + upd_w(pn)))

    wd = params["double"]["W"].T
    bd = params["double"]["b"].reshape(1, 2 * H)
    w1c = jnp.transpose(params["conv1"]["W"], (1, 2, 0)).reshape(2 * K1, 8)
    b1c = params["conv1"]["b"].reshape(1, 8)
    w2c = jnp.transpose(params["conv2"]["W"], (2, 1, 0)).reshape(8 * K2, 2)
    b2c = params["conv2"]["b"].reshape(1, 2)

    cntp = _degree(dst2)
    h, cinv = _embed(z, cntp, w1e, b1e, w2e, b2e)
    for i in range(NL):
        whA, whB, wzA, wzB, bA, w2g, b2g, w2n, b2n, uw = layer_w[i]
        a_tab, b_tab = _prep(h, z, whA, whB, wzA, wzB, bA)
        p_edge = _gather_pairs(a_tab, b_tab, dst2, src2)
        m_edge = _edge_mlp(p_edge, w2g, b2g, w2n, b2n)
        agg = _scatter_sum(m_edge, dst2)
        h = _update(h, agg[0], agg[1], cinv, variables, uw)
    return _decode(h, x, wd, bd, w1c, b1c, w2c, b2c)


# speedup vs baseline: 1.4197x; 1.3221x over previous
import functools

import jax
import jax.numpy as jnp
from jax import lax
from jax.experimental import pallas as pl
from jax.experimental.pallas import tpu as pltpu
from jax.experimental.pallas import tpu_sc as plsc

N = 10000
E = 160000
TW = 25
H = 128
NL = 6
L_PDE = 16.0
TMAX = 4.0
DT = 0.16

F32 = jnp.float32

NC = 2
NS = 16
NW = NC * NS
CH = 128
CHN = E // CH
CHP = 1280
EP = CHP * CH
GB = CHN // NW
GX = CHN % NW
GBU = CHP // NW
SBU = CHP // NS
DTN = 10
DRW = N // DTN
ZRW = 40

def _swish(x):
    return x * (1.0 / (1.0 + jnp.exp(-x)))


def _sigmoid(x):
    return 1.0 / (1.0 + jnp.exp(-x))


@functools.cache
def _sc_kernels():
    mesh = plsc.VectorSubcoreMesh(core_axis_name="c", subcore_axis_name="s",
                                  num_cores=NC, num_subcores=NS)
    gather = functools.partial(
        pl.kernel,
        out_type=jax.ShapeDtypeStruct((EP, 2 * H), F32),
        mesh=mesh,
        scratch_types=[
            pltpu.VMEM((GBU, 1, CH), jnp.int32),
            pltpu.VMEM((GBU, 1, CH), jnp.int32),
            pltpu.VMEM((CH, 2 * H), F32),
            pltpu.VMEM((CH, 2 * H), F32),
            pltpu.SemaphoreType.DMA,
            pltpu.SemaphoreType.DMA,
        ],
    )(_gather_body)
    scatter = functools.partial(
        pl.kernel,
        out_type=jax.ShapeDtypeStruct((NC, N, H), F32),
        mesh=mesh,
        scratch_types=[
            pltpu.VMEM((SBU, 1, CH), jnp.int32),
            pltpu.VMEM((CH, H), F32),
            pltpu.VMEM((CH, H), F32),
            pltpu.VMEM((ZRW, H), F32),
            pltpu.VMEM_SHARED((N, H), F32),
            pltpu.SemaphoreType.DMA,
            pltpu.SemaphoreType.DMA,
        ],
    )(_scatter_body)
    degree = functools.partial(
        pl.kernel,
        out_type=jax.ShapeDtypeStruct((NC, N, H), F32),
        mesh=mesh,
        scratch_types=[
            pltpu.VMEM((GB + 1, 1, CH), jnp.int32),
            pltpu.VMEM((CH, H), F32),
            pltpu.VMEM((ZRW, H), F32),
            pltpu.VMEM_SHARED((N, H), F32),
        ],
    )(_degree_body)
    return gather, scatter, degree


def _gather_pairs(a_tab, b_tab, dst2, src2):
    return _sc_kernels()[0](a_tab, b_tab, dst2, src2)


def _scatter_sum(m_edge, dst2):
    return _sc_kernels()[1](m_edge, dst2)


def _degree(dst2):
    return _sc_kernels()[2](dst2)


def _gather_body(a_hbm, b_hbm, dst_hbm, src_hbm, out_hbm,
                 dst_v, src_v, ra, rb, sem_a, sem_b):
    wid = lax.axis_index("s") * NC + lax.axis_index("c")
    start = GBU * wid
    pltpu.sync_copy(dst_hbm.at[pl.ds(start, GBU)], dst_v)
    pltpu.sync_copy(src_hbm.at[pl.ds(start, GBU)], src_v)

    def body(j, carry):
        ca = pltpu.async_copy(a_hbm.at[dst_v.at[j, 0]], ra, sem_a)
        cb = pltpu.async_copy(b_hbm.at[src_v.at[j, 0]], rb, sem_b)
        ca.wait()
        cb.wait()

        def add_row(r, c):
            for blk in range(2 * H // 16):
                sl = pl.ds(blk * 16, 16)
                ra[r, sl] = ra[r, sl] + rb[r, sl]
            return c

        lax.fori_loop(0, CH, add_row, 0)
        pltpu.sync_copy(ra, out_hbm.at[pl.ds((start + j) * CH, CH), :])
        return carry

    lax.fori_loop(0, GBU, body, 0)


def _scatter_body(m_hbm, dst_hbm, out_hbm, dst_v, m0, m1, zbuf, acc,
                  sem0, sem1):
    cid = lax.axis_index("c")
    sid = lax.axis_index("s")
    start = SBU * sid
    pltpu.sync_copy(dst_hbm.at[pl.ds(start, SBU)], dst_v)

    def zrow(r, c):
        for blk in range(H // 16):
            zbuf[r, pl.ds(blk * 16, 16)] = jnp.zeros((16,), F32)
        return c

    lax.fori_loop(0, ZRW, zrow, 0)

    @pl.when(sid < DTN)
    def _():
        for k in range(DRW // ZRW):
            pltpu.sync_copy(zbuf, acc.at[pl.ds(sid * DRW + k * ZRW, ZRW), :])

    plsc.subcore_barrier()

    def load(j, buf, sem):
        return pltpu.async_copy(
            m_hbm.at[pl.ds((start + j) * CH, CH), pl.ds(cid * H, H)], buf, sem)

    def drain(buf, sem):
        pltpu.make_async_copy(
            m_hbm.at[pl.ds(0, CH), pl.ds(0, H)], buf, sem).wait()

    load(0, m0, sem0)
    npairs = SBU // 2

    def body(jp, carry):
        j0 = 2 * jp
        load(j0 + 1, m1, sem1)
        drain(m0, sem0)
        pltpu.sync_copy(m0, acc.at[dst_v.at[j0, 0]], add=True)

        @pl.when(jp + 1 < npairs)
        def _():
            load(j0 + 2, m0, sem0)

        drain(m1, sem1)
        pltpu.sync_copy(m1, acc.at[dst_v.at[j0 + 1, 0]], add=True)
        return carry

    lax.fori_loop(0, npairs, body, 0)
    plsc.subcore_barrier()

    @pl.when(sid < DTN)
    def _():
        sl = pl.ds(sid * DRW, DRW)
        pltpu.sync_copy(acc.at[sl, :], out_hbm.at[cid, sl, :])


def _degree_body(dst_hbm, out_hbm, dst_v, ones_v, zbuf, acc):
    cid = lax.axis_index("c")
    sid = lax.axis_index("s")
    wid = sid * NC + cid
    start = GB * wid + jnp.minimum(wid, GX)
    nch = GB + jnp.where(wid < GX, 1, 0)
    pltpu.sync_copy(dst_hbm.at[pl.ds(start, GB + 1)], dst_v)

    def fill(r, c):
        for blk in range(H // 16):
            ones_v[r, pl.ds(blk * 16, 16)] = jnp.ones((16,), F32)
        return c

    lax.fori_loop(0, CH, fill, 0)

    def zfill(r, c):
        for blk in range(H // 16):
            zbuf[r, pl.ds(blk * 16, 16)] = jnp.zeros((16,), F32)
        return c

    lax.fori_loop(0, ZRW, zfill, 0)

    @pl.when(sid < DTN)
    def _():
        for k in range(DRW // ZRW):
            pltpu.sync_copy(zbuf, acc.at[pl.ds(sid * DRW + k * ZRW, ZRW), :])

    plsc.subcore_barrier()

    def body(j, carry):
        pltpu.sync_copy(ones_v, acc.at[dst_v.at[j, 0]], add=True)
        return carry

    lax.fori_loop(0, nch, body, 0)
    plsc.subcore_barrier()

    @pl.when(sid < DTN)
    def _():
        sl = pl.ds(sid * DRW, DRW)
        pltpu.sync_copy(acc.at[sl, :], out_hbm.at[cid, sl, :])


def _embed_body(z_ref, cntp_ref, w1_ref, b1_ref, w2_ref, b2_ref,
                h_ref, cinv_ref):
    z = z_ref[...]
    h0 = _swish(jnp.dot(z, w1_ref[...], preferred_element_type=F32)
                + b1_ref[...])
    h_ref[...] = _swish(jnp.dot(h0, w2_ref[...], preferred_element_type=F32)
                        + b2_ref[...])
    cnt = cntp_ref[0, :, 0:1] + cntp_ref[1, :, 0:1]
    cinv_ref[...] = 1.0 / jnp.maximum(cnt, 1.0)


def _embed(z, cntp, w1, b1, w2, b2):
    return pl.pallas_call(
        _embed_body,
        out_shape=[jax.ShapeDtypeStruct((N, H), F32),
                   jax.ShapeDtypeStruct((N, 1), F32)],
    )(z, cntp, w1, b1, w2, b2)


NBK = 1000


def _prep_body(h_ref, z_ref, whA_ref, whB_ref, wzA_ref, wzB_ref, bA_ref,
               a_ref, b_ref):
    h = h_ref[...]
    z = z_ref[...]
    a_ref[...] = (jnp.dot(h, whA_ref[...], preferred_element_type=F32)
                  + jnp.dot(z, wzA_ref[...], preferred_element_type=F32)
                  + bA_ref[...])
    b_ref[...] = (jnp.dot(h, whB_ref[...], preferred_element_type=F32)
                  + jnp.dot(z, wzB_ref[...], preferred_element_type=F32))


def _prep(h, z, whA, whB, wzA, wzB, bA):
    nb = N // NBK
    blk = lambda i: (i, 0)
    zero = lambda i: (0, 0)
    return pl.pallas_call(
        _prep_body,
        grid=(nb,),
        in_specs=[
            pl.BlockSpec((NBK, H), blk),
            pl.BlockSpec((NBK, 52), blk),
            pl.BlockSpec((H, 2 * H), zero),
            pl.BlockSpec((H, 2 * H), zero),
            pl.BlockSpec((52, 2 * H), zero),
            pl.BlockSpec((52, 2 * H), zero),
            pl.BlockSpec((1, 2 * H), zero),
        ],
        out_specs=[pl.BlockSpec((NBK, 2 * H), blk),
                   pl.BlockSpec((NBK, 2 * H), blk)],
        out_shape=[jax.ShapeDtypeStruct((N, 2 * H), F32),
                   jax.ShapeDtypeStruct((N, 2 * H), F32)],
    )(h, z, whA, whB, wzA, wzB, bA)


EBK = 1024


def _edge_body(p_ref, wg_ref, bg_ref, wn_ref, bn_ref, m_ref):
    rows = (pl.program_id(0) * EBK
            + lax.broadcasted_iota(jnp.int32, (EBK, 1), 0))
    mask = rows < E
    s = _swish(p_ref[...])
    mg = _swish(jnp.dot(s[:, :H], wg_ref[...], preferred_element_type=F32)
                + bg_ref[...])
    mn = _swish(jnp.dot(s[:, H:], wn_ref[...], preferred_element_type=F32)
                + bn_ref[...])
    m_ref[:, :H] = jnp.where(mask, mg, 0.0)
    m_ref[:, H:] = jnp.where(mask, mn, 0.0)


def _edge_mlp(p, wg, bg, wn, bn):
    nb = EP // EBK
    blk = lambda i: (i, 0)
    zero = lambda i: (0, 0)
    return pl.pallas_call(
        _edge_body,
        grid=(nb,),
        in_specs=[
            pl.BlockSpec((EBK, 2 * H), blk),
            pl.BlockSpec((H, H), zero),
            pl.BlockSpec((1, H), zero),
            pl.BlockSpec((H, H), zero),
            pl.BlockSpec((1, H), zero),
        ],
        out_specs=pl.BlockSpec((EBK, 2 * H), blk),
        out_shape=jax.ShapeDtypeStruct((EP, 2 * H), F32),
    )(p, wg, bg, wn, bn)


def _update_body(h_ref, ag_ref, an_ref, cinv_ref, var_ref,
                 whg_ref, wag_ref, wvg_ref, b1g_ref, w2g_ref, b2g_ref,
                 whn_ref, wan_ref, wvn_ref, b1n_ref, w2n_ref, b2n_ref,
                 hn_ref):
    h = h_ref[...]
    cinv = cinv_ref[...]
    var = var_ref[...]

    def half(ag, wh, wa, wv, b1, w2, b2):
        mean = ag * cinv
        t = _swish(jnp.dot(h, wh, preferred_element_type=F32)
                   + jnp.dot(mean, wa, preferred_element_type=F32)
                   + var * wv + b1)
        upd = jnp.dot(t, w2, preferred_element_type=F32) + b2
        out = h + upd
        mu = jnp.mean(out, axis=0, keepdims=True)
        d = out - mu
        v = jnp.mean(d * d, axis=0, keepdims=True)
        return d * lax.rsqrt(v + 1e-5)

    ngate = half(ag_ref[...], whg_ref[...], wag_ref[...], wvg_ref[...],
                 b1g_ref[...], w2g_ref[...], b2g_ref[...])
    ngnn = half(an_ref[...], whn_ref[...], wan_ref[...], wvn_ref[...],
                b1n_ref[...], w2n_ref[...], b2n_ref[...])
    tau = _sigmoid(ngate)
    g = _swish(ngnn)
    hn_ref[...] = (1.0 - tau) * h + tau * g


def _update(h, ag, an, cinv, var, wts):
    return pl.pallas_call(
        _update_body,
        out_shape=jax.ShapeDtypeStruct((N, H), F32),
    )(h, ag, an, cinv, var, *wts)


NC1 = 38
K1 = 16
S1 = 3
K2 = 14


def _dec_body(h_ref, u_ref, wd_ref, bd_ref, w1_ref, b1_ref, w2_ref, b2_ref,
              o_ref, d1_ref):
    hd = _swish(jnp.dot(h_ref[...], wd_ref[...], preferred_element_type=F32)
                + bd_ref[...])
    for t in range(NC1):
        x1 = jnp.concatenate(
            [hd[:, S1 * t:S1 * t + K1], hd[:, H + S1 * t:H + S1 * t + K1]],
            axis=1)
        d1_ref[:, t * 8:(t + 1) * 8] = _swish(
            jnp.dot(x1, w1_ref[...], preferred_element_type=F32) + b1_ref[...])
    for t in range(TW):
        x2 = d1_ref[:, t * 8:t * 8 + 8 * K2]
        d2 = jnp.dot(x2, w2_ref[...], preferred_element_type=F32) + b2_ref[...]
        dt = DT * (t + 1)
        o_ref[:, pl.ds(t, 1)] = u_ref[:, pl.ds(t, 1)] + dt * d2[:, 0:1]
        o_ref[:, pl.ds(TW + t, 1)] = u_ref[:, pl.ds(TW + t, 1)] + dt * d2[:, 1:2]


def _decode(h, u, wd, bd, w1, b1, w2, b2):
    nb = N // NBK
    blk = lambda i: (i, 0)
    zero = lambda i: (0, 0)
    return pl.pallas_call(
        _dec_body,
        grid=(nb,),
        in_specs=[
            pl.BlockSpec((NBK, H), blk),
            pl.BlockSpec((NBK, 2 * TW), blk),
            pl.BlockSpec((H, 2 * H), zero),
            pl.BlockSpec((1, 2 * H), zero),
            pl.BlockSpec((2 * K1, 8), zero),
            pl.BlockSpec((1, 8), zero),
            pl.BlockSpec((8 * K2, 2), zero),
            pl.BlockSpec((1, 2), zero),
        ],
        out_specs=pl.BlockSpec((NBK, 2 * TW), blk),
        out_shape=jax.ShapeDtypeStruct((N, 2 * TW), F32),
        scratch_shapes=[pltpu.VMEM((NBK, 8 * NC1), F32)],
    )(h, u, wd, bd, w1, b1, w2, b2)


def kernel(x, pos, edge_index, batch, params):
    del batch
    pad = (jnp.arange((CHP - CHN) * CH, dtype=jnp.int32) % N).reshape(
        CHP - CHN, CH)
    src2 = jnp.concatenate([edge_index[0].reshape(CHN, CH), pad]
                           ).reshape(CHP, 1, CH)
    dst2 = jnp.concatenate([edge_index[1].reshape(CHN, CH), pad]
                           ).reshape(CHP, 1, CH)
    pos_x = pos[:, 1:2] / L_PDE
    variables = pos[:, 0:1] / TMAX
    z = jnp.concatenate([x, pos_x, variables], axis=1)

    w1e = params["emb1"]["W"].T
    b1e = params["emb1"]["b"].reshape(1, H)
    w2e = params["emb2"]["W"].T
    b2e = params["emb2"]["b"].reshape(1, H)

    layer_w = []
    for i in range(NL):
        pg = params["gate"][i]
        pn = params["gnn"][i]
        Wg = pg["msg1"]["W"]
        Wn = pn["msg1"]["W"]
        whA = jnp.concatenate([Wg[:, :H].T, Wn[:, :H].T], axis=1)
        whB = jnp.concatenate([Wg[:, H:2 * H].T, Wn[:, H:2 * H].T], axis=1)
        wzA = jnp.concatenate([Wg[:, 2 * H:].T, Wn[:, 2 * H:].T], axis=1)
        mask = jnp.ones((52, 1), F32).at[51, 0].set(0.0)
        wzB = jnp.concatenate([-Wg[:, 2 * H:].T * mask,
                               -Wn[:, 2 * H:].T * mask], axis=1)
        bA = jnp.concatenate([pg["msg1"]["b"], pn["msg1"]["b"]]).reshape(1, 2 * H)
        w2g = pg["msg2"]["W"].T
        b2g = pg["msg2"]["b"].reshape(1, H)
        w2n = pn["msg2"]["W"].T
        b2n = pn["msg2"]["b"].reshape(1, H)

        def upd_w(p):
            U1 = p["upd1"]["W"]
            return (U1[:, :H].T, U1[:, H:2 * H].T, U1[:, 2 * H:].T,
                    p["upd1"]["b"].reshape(1, H), p["upd2"]["W"].T,
                    p["upd2"]["b"].reshape(1, H))

        layer_w.append((whA, whB, wzA, wzB, bA, w2g, b2g, w2n, b2n,
                        upd_w(pg) + upd_w(pn)))

    wd = params["double"]["W"].T
    bd = params["double"]["b"].reshape(1, 2 * H)
    w1c = jnp.transpose(params["conv1"]["W"], (1, 2, 0)).reshape(2 * K1, 8)
    b1c = params["conv1"]["b"].reshape(1, 8)
    w2c = jnp.transpose(params["conv2"]["W"], (2, 1, 0)).reshape(8 * K2, 2)
    b2c = params["conv2"]["b"].reshape(1, 2)

    cntp = _degree(dst2)
    h, cinv = _embed(z, cntp, w1e, b1e, w2e, b2e)
    for i in range(NL):
        whA, whB, wzA, wzB, bA, w2g, b2g, w2n, b2n, uw = layer_w[i]
        a_tab, b_tab = _prep(h, z, whA, whB, wzA, wzB, bA)
        p_edge = _gather_pairs(a_tab, b_tab, dst2, src2)
        m_edge = _edge_mlp(p_edge, w2g, b2g, w2n, b2n)
        agg = _scatter_sum(m_edge, dst2)
        h = _update(h, agg[0], agg[1], cinv, variables, uw)
    return _decode(h, x, wd, bd, w1c, b1c, w2c, b2c)
